# baseline (speedup 1.0000x reference)
import functools

import jax
import jax.numpy as jnp
from jax import lax
from jax.experimental import pallas as pl
from jax.experimental.pallas import tpu as pltpu
from jax.experimental.pallas import tpu_sc as plsc

B, S = 4096, 50
NG = 32
NQ = 64
D = 128
T = B * S
NW = 32
TPW = T // NW
CH = 64
NCH = TPW // CH
L = 16


BB = 128
G = B // BB
IR = BB * S // 128


def _decode_body(g_ref, q_ref, i0_ref, i1_ref):
    iota_g = lax.broadcasted_iota(
        jnp.int32, (1, 1, NG), 2).astype(jnp.float32)
    gs = jnp.sum(g_ref[...] * iota_g, axis=2)
    iota_q = lax.broadcasted_iota(
        jnp.int32, (1, 1, 1, NQ), 3).astype(jnp.float32)
    qs = jnp.sum(q_ref[...] * iota_q, axis=3)
    i0 = NQ * gs + qs[:, :, 0]
    i1 = (NG * NQ) + NQ * gs + qs[:, :, 1]
    i0_ref[...] = i0.astype(jnp.int32).reshape(1, IR, 128)
    i1_ref[...] = i1.astype(jnp.int32).reshape(1, IR, 128)


_decode = pl.pallas_call(
    _decode_body,
    grid=(G,),
    in_specs=[
        pl.BlockSpec((BB, S, NG), lambda i: (i, 0, 0)),
        pl.BlockSpec((BB, S, 2, NQ), lambda i: (i, 0, 0, 0)),
    ],
    out_specs=[
        pl.BlockSpec((1, IR, 128), lambda i: (i, 0, 0)),
        pl.BlockSpec((1, IR, 128), lambda i: (i, 0, 0)),
    ],
    out_shape=[jax.ShapeDtypeStruct((G, IR, 128), jnp.int32)] * 2,
)


def _table_body(w_ref, qt_ref, ct_ref):
    w = w_ref[...]
    qt = qt_ref[...]
    lo = w[:, None, :D] + qt[None, :, :]
    hi = w[:, None, D:] + qt[None, :, :]
    ct_ref[...] = jnp.concatenate(
        [lo.reshape(NG * NQ, D), hi.reshape(NG * NQ, D)], axis=0)


_build_table = pl.pallas_call(
    _table_body,
    out_shape=jax.ShapeDtypeStruct((2 * NG * NQ, D), jnp.float32),
)


_mesh = plsc.VectorSubcoreMesh(core_axis_name="c", subcore_axis_name="s")


@functools.partial(
    pl.kernel,
    out_type=jax.ShapeDtypeStruct((2 * T, D), jnp.float32),
    mesh=_mesh,
    compiler_params=pltpu.CompilerParams(needs_layout_passes=False),
    scratch_types=(
        [pltpu.VMEM((TPW,), jnp.int32)] * 2
        + [pltpu.VMEM((2 * CH,), jnp.int32)] * 4
        + [pltpu.VMEM((2 * CH, D), jnp.float32)] * 4
        + [pltpu.SemaphoreType.DMA] * 8
    ),
)
def _sc_gather(i0_hbm, i1_hbm, ct_hbm, out_hbm,
               i0_v, i1_v, ii0, ii1, ii2, ii3, ob0, ob1, ob2, ob3,
               gsem0, gsem1, gsem2, gsem3, osem0, osem1, osem2, osem3):
    wid = lax.axis_index("s") * 2 + lax.axis_index("c")
    base = wid * TPW
    pltpu.sync_copy(i0_hbm.at[pl.ds(base, TPW)], i0_v)
    pltpu.sync_copy(i1_hbm.at[pl.ds(base, TPW)], i1_v)

    iis = (ii0, ii1, ii2, ii3)
    obs = (ob0, ob1, ob2, ob3)
    gsems = (gsem0, gsem1, gsem2, gsem3)
    osems = (osem0, osem1, osem2, osem3)
    iota = lax.iota(jnp.int32, L)

    def start_gather(ci, b):
        off = ci * CH
        for m in range(CH // L):
            sl = pl.ds(off + m * L, L)
            plsc.store_scatter(iis[b], [2 * (m * L + iota)], i0_v[sl])
            plsc.store_scatter(iis[b], [2 * (m * L + iota) + 1], i1_v[sl])
        pltpu.async_copy(ct_hbm.at[iis[b]], obs[b], gsems[b])

    def finish_chunk(ci, b):
        pltpu.make_async_copy(ct_hbm.at[iis[b]], obs[b], gsems[b]).wait()
        dst = out_hbm.at[pl.ds(2 * (base + ci * CH), 2 * CH)]
        pltpu.async_copy(obs[b], dst, osems[b])

    def drain_out(ci, b):
        dst = out_hbm.at[pl.ds(2 * (base + ci * CH), 2 * CH)]
        pltpu.make_async_copy(obs[b], dst, osems[b]).wait()

    NB = 4

    def block(li, carry):
        for b in range(NB):
            ci = NB * li + b

            @pl.when(li > 0)
            def _reclaim():
                drain_out(ci - NB, b)

            start_gather(ci, b)
            if b == 0:

                @pl.when(li > 0)
                def _finish_prev():
                    finish_chunk(ci - 1, NB - 1)

            else:
                finish_chunk(ci - 1, b - 1)
        return carry

    lax.fori_loop(0, NCH // NB, block, 0)
    finish_chunk(NCH - 1, NB - 1)
    for b in range(NB):
        drain_out(NCH - NB + b, b)


def kernel(gates_oh, gate_qubits_oh, qubits, W_gate):
    i0, i1 = _decode(gates_oh, gate_qubits_oh)
    ct = _build_table(W_gate, qubits)
    out = _sc_gather(i0.reshape(T), i1.reshape(T), ct)
    return out.reshape(B, S, 2 * D)


# speedup vs baseline: 1.0852x; 1.0852x over previous
import functools

import jax
import jax.numpy as jnp
from jax import lax
from jax.experimental import pallas as pl
from jax.experimental.pallas import tpu as pltpu
from jax.experimental.pallas import tpu_sc as plsc

B, S = 4096, 50
NG = 32
NQ = 64
D = 128
T = B * S
NW = 32
TPW = T // NW
CH = 64
NCH = TPW // CH
L = 16


RB = 2048
G = T // RB
IR = RB // 128


def _decode_body(g_ref, q_ref, i0_ref, i1_ref):
    iota_g = lax.broadcasted_iota(jnp.int32, (1, NG), 1).astype(jnp.float32)
    gs = jnp.sum(g_ref[...] * iota_g, axis=1)
    iota_q = lax.broadcasted_iota(
        jnp.int32, (1, 1, NQ), 2).astype(jnp.float32)
    qs = jnp.sum(q_ref[...] * iota_q, axis=2)
    i0 = NQ * gs + qs[:, 0]
    i1 = (NG * NQ) + NQ * gs + qs[:, 1]
    i0_ref[...] = i0.astype(jnp.int32).reshape(IR, 128)
    i1_ref[...] = i1.astype(jnp.int32).reshape(IR, 128)


_decode = pl.pallas_call(
    _decode_body,
    grid=(G,),
    in_specs=[
        pl.BlockSpec((RB, NG), lambda i: (i, 0)),
        pl.BlockSpec((RB, 2, NQ), lambda i: (i, 0, 0)),
    ],
    out_specs=[
        pl.BlockSpec((IR, 128), lambda i: (i, 0)),
        pl.BlockSpec((IR, 128), lambda i: (i, 0)),
    ],
    out_shape=[jax.ShapeDtypeStruct((T // 128, 128), jnp.int32)] * 2,
)


def _table_body(w_ref, qt_ref, ct_ref):
    w = w_ref[...]
    qt = qt_ref[...]
    lo = w[:, None, :D] + qt[None, :, :]
    hi = w[:, None, D:] + qt[None, :, :]
    ct_ref[...] = jnp.concatenate(
        [lo.reshape(NG * NQ, D), hi.reshape(NG * NQ, D)], axis=0)


_build_table = pl.pallas_call(
    _table_body,
    out_shape=jax.ShapeDtypeStruct((2 * NG * NQ, D), jnp.float32),
)


_mesh = plsc.VectorSubcoreMesh(core_axis_name="c", subcore_axis_name="s")


@functools.partial(
    pl.kernel,
    out_type=jax.ShapeDtypeStruct((2 * T, D), jnp.float32),
    mesh=_mesh,
    compiler_params=pltpu.CompilerParams(needs_layout_passes=False),
    scratch_types=(
        [pltpu.VMEM((TPW,), jnp.int32)] * 2
        + [pltpu.VMEM((2 * CH,), jnp.int32)] * 4
        + [pltpu.VMEM((2 * CH, D), jnp.float32)] * 4
        + [pltpu.SemaphoreType.DMA] * 8
    ),
)
def _sc_gather(i0_hbm, i1_hbm, ct_hbm, out_hbm,
               i0_v, i1_v, ii0, ii1, ii2, ii3, ob0, ob1, ob2, ob3,
               gsem0, gsem1, gsem2, gsem3, osem0, osem1, osem2, osem3):
    wid = lax.axis_index("s") * 2 + lax.axis_index("c")
    base = wid * TPW
    pltpu.sync_copy(i0_hbm.at[pl.ds(base, TPW)], i0_v)
    pltpu.sync_copy(i1_hbm.at[pl.ds(base, TPW)], i1_v)

    iis = (ii0, ii1, ii2, ii3)
    obs = (ob0, ob1, ob2, ob3)
    gsems = (gsem0, gsem1, gsem2, gsem3)
    osems = (osem0, osem1, osem2, osem3)
    iota = lax.iota(jnp.int32, L)

    def start_gather(ci, b):
        off = ci * CH
        for m in range(CH // L):
            sl = pl.ds(off + m * L, L)
            plsc.store_scatter(iis[b], [2 * (m * L + iota)], i0_v[sl])
            plsc.store_scatter(iis[b], [2 * (m * L + iota) + 1], i1_v[sl])
        pltpu.async_copy(ct_hbm.at[iis[b]], obs[b], gsems[b])

    def finish_chunk(ci, b):
        pltpu.make_async_copy(ct_hbm.at[iis[b]], obs[b], gsems[b]).wait()
        dst = out_hbm.at[pl.ds(2 * (base + ci * CH), 2 * CH)]
        pltpu.async_copy(obs[b], dst, osems[b])

    def drain_out(ci, b):
        dst = out_hbm.at[pl.ds(2 * (base + ci * CH), 2 * CH)]
        pltpu.make_async_copy(obs[b], dst, osems[b]).wait()

    NB = 4

    def block(li, carry):
        for b in range(NB):
            ci = NB * li + b

            @pl.when(li > 0)
            def _reclaim():
                drain_out(ci - NB, b)

            start_gather(ci, b)
            if b == 0:

                @pl.when(li > 0)
                def _finish_prev():
                    finish_chunk(ci - 1, NB - 1)

            else:
                finish_chunk(ci - 1, b - 1)
        return carry

    lax.fori_loop(0, NCH // NB, block, 0)
    finish_chunk(NCH - 1, NB - 1)
    for b in range(NB):
        drain_out(NCH - NB + b, b)


def kernel(gates_oh, gate_qubits_oh, qubits, W_gate):
    g_flat = gates_oh.reshape(T, NG)
    q_flat = gate_qubits_oh.reshape(T, 2, NQ)
    i0, i1 = _decode(g_flat, q_flat)
    ct = _build_table(W_gate, qubits)
    out = _sc_gather(i0.reshape(T), i1.reshape(T), ct)
    return out.reshape(B, S, 2 * D)
